# Initial kernel scaffold; baseline (speedup 1.0000x reference)
import jax
import jax.numpy as jnp
from jax.experimental import pallas as pl


def kernel(x, edge_index, batch, W1, b1, W2, b2, gamma, beta, Wd1, bd1, Wd2, bd2):
    raise NotImplementedError("write your pallas kernel here")


# speedup vs baseline: 5.0640x; 5.0640x over previous
import functools

import jax
import jax.numpy as jnp
from jax import lax
from jax.experimental import pallas as pl
from jax.experimental.pallas import tpu as pltpu
from jax.experimental.pallas import tpu_sc as plsc

N = 10000
E = 320000
D = 128
NCLS = 10
L = 8
G = 64
NEG_SLOPE = 0.01

NC = 2
NS = 16
NW = NC * NS

CH = 128
EPT = 10112
EP = NW * EPT
NCH = EPT // CH
NPAD = 10240
RPT = NPAD // NS


def _make_seg_sum():
    mesh = plsc.VectorSubcoreMesh(
        core_axis_name="c", subcore_axis_name="s", num_cores=NC, num_subcores=NS
    )

    @functools.partial(
        pl.kernel,
        out_type=jax.ShapeDtypeStruct((NC * NPAD, D), jnp.float32),
        mesh=mesh,
        scratch_types=[
            pltpu.VMEM((CH,), jnp.int32),
            pltpu.VMEM((CH,), jnp.int32),
            pltpu.VMEM((CH, D), jnp.float32),
            pltpu.VMEM_SHARED((NPAD, D), jnp.float32),
            pltpu.SemaphoreType.DMA,
        ],
    )
    def seg_sum(h_hbm, src_hbm, dst_hbm, z_hbm, out_hbm,
                src_v, dst_v, rows_v, acc_sh, sem):
        cid = lax.axis_index("c")
        sid = lax.axis_index("s")
        tile = cid * NS + sid

        pltpu.sync_copy(z_hbm, acc_sh.at[pl.ds(sid * RPT, RPT)])
        plsc.subcore_barrier()

        ebase = tile * EPT

        def body(i, carry):
            off = ebase + i * CH
            pltpu.sync_copy(src_hbm.at[pl.ds(off, CH)], src_v)
            pltpu.sync_copy(dst_hbm.at[pl.ds(off, CH)], dst_v)
            pltpu.async_copy(h_hbm.at[src_v], rows_v, sem).wait()
            pltpu.sync_copy(rows_v, acc_sh.at[dst_v], add=True)
            return carry

        lax.fori_loop(0, NCH, body, 0)
        plsc.subcore_barrier()

        pltpu.sync_copy(
            acc_sh.at[pl.ds(sid * RPT, RPT)],
            out_hbm.at[pl.ds(cid * NPAD + sid * RPT, RPT)],
        )

    return seg_sum


_seg_sum_cache = []


def _seg_sum(h, src_p, dst_p, zeros):
    if not _seg_sum_cache:
        _seg_sum_cache.append(_make_seg_sum())
    return _seg_sum_cache[0](h, src_p, dst_p, zeros)


def _dense_body(h_ref, a0_ref, a1_ref, w1_ref, b1_ref, w2_ref, b2_ref,
                g_ref, be_ref, o_ref, *, last):
    z = h_ref[...] + a0_ref[...] + a1_ref[...]
    z = jnp.dot(z, w1_ref[...], preferred_element_type=jnp.float32, precision=lax.Precision.HIGHEST) + b1_ref[...]
    z = jnp.where(z > 0, z, NEG_SLOPE * z)
    y = jnp.dot(z, w2_ref[...], preferred_element_type=jnp.float32, precision=lax.Precision.HIGHEST) + b2_ref[...]
    mu = jnp.mean(y, axis=0, keepdims=True)
    yc = y - mu
    var = jnp.mean(yc * yc, axis=0, keepdims=True)
    out = yc * lax.rsqrt(var + 1e-5) * g_ref[...] + be_ref[...]
    if not last:
        out = jnp.where(out > 0, out, NEG_SLOPE * out)
    o_ref[...] = out


def _dense_layer(h, a0, a1, w1, b1, w2, b2, g, be, last):
    return pl.pallas_call(
        functools.partial(_dense_body, last=last),
        out_shape=jax.ShapeDtypeStruct((N, D), jnp.float32),
    )(h, a0, a1, w1, b1, w2, b2, g, be)


def _final_body(h_ref, u_ref, brow_ref, wd1_ref, bd1_ref, wd2_ref, bd2_ref,
                probs_ref, c_ref):
    u = u_ref[...]
    y = h_ref[...] + (-jnp.log(-jnp.log(u)))
    m = jnp.max(y, axis=1, keepdims=True)
    ex = jnp.exp(y - m)
    c = ex / jnp.sum(ex, axis=1, keepdims=True)
    c_ref[...] = c
    ohT = (lax.broadcasted_iota(jnp.int32, (G, N), 0) == brow_ref[...]).astype(
        jnp.float32)
    pooled = jnp.dot(ohT, c, preferred_element_type=jnp.float32, precision=lax.Precision.HIGHEST)
    d = jnp.dot(pooled, wd1_ref[...], preferred_element_type=jnp.float32, precision=lax.Precision.HIGHEST) + bd1_ref[...]
    d = jnp.where(d > 0, d, NEG_SLOPE * d)
    e2 = jnp.dot(d, wd2_ref[...], preferred_element_type=jnp.float32, precision=lax.Precision.HIGHEST) + bd2_ref[...]
    m2 = jnp.max(e2, axis=1, keepdims=True)
    ex2 = jnp.exp(e2 - m2)
    probs_ref[...] = ex2 / jnp.sum(ex2, axis=1, keepdims=True)


def _final(h, u, brow, wd1, bd1, wd2, bd2):
    return pl.pallas_call(
        _final_body,
        out_shape=(
            jax.ShapeDtypeStruct((G, NCLS), jnp.float32),
            jax.ShapeDtypeStruct((N, D), jnp.float32),
        ),
    )(h, u, brow, wd1, bd1, wd2, bd2)


def kernel(x, edge_index, batch, W1, b1, W2, b2, gamma, beta, Wd1, bd1, Wd2, bd2):
    src = edge_index[0]
    dst = edge_index[1]
    pad = EP - E
    apad = jnp.arange(pad, dtype=jnp.int32)
    src_p = jnp.concatenate([src, apad % N])
    dst_p = jnp.concatenate([dst, N + apad % (NPAD - N)])
    zeros = jnp.zeros((RPT, D), jnp.float32)
    brow = batch[None, :]
    u = jax.random.uniform(jax.random.key(42), (N, D), minval=1e-6, maxval=1.0 - 1e-6)

    h = x
    for i in range(L):
        agg2 = _seg_sum(h, src_p, dst_p, zeros)
        h = _dense_layer(h, agg2[:N], agg2[NPAD:NPAD + N],
                         W1[i], b1[i][None, :], W2[i], b2[i][None, :],
                         gamma[i][None, :], beta[i][None, :], last=(i == L - 1))
    probs, c = _final(h, u, brow, Wd1, bd1[None, :], Wd2, bd2[None, :])
    return probs, c
